# baseline (speedup 1.0000x reference)
import jax
import jax.numpy as jnp
from jax import lax
from jax.experimental import pallas as pl
from jax.experimental.pallas import tpu as pltpu
from jax.experimental.pallas import tpu_sc as plsc

NUM_Q = 900
NUM_C = 91
N_REAL = NUM_Q * NUM_C
N_BUF = 81920
NV = N_BUF // 16
N_DMA = 81904
K_OUT = 300
K_PAD = 304
CAP = 4096
ROWS_PER_W = 4

_I32 = jnp.int32
_U32 = jnp.uint32


def _lane():
    return lax.iota(_I32, 16)


def _popcount(mask):
    return plsc.all_reduce_population_count(mask)[0]


def _keys_of(bits):
    m = plsc.bitcast(plsc.bitcast(bits, _I32) >> 31, _U32)
    return bits ^ (m | jnp.uint32(0x80000000))


def _tk_body(bits_hbm, boxes_hbm, wh_hbm, scores_hbm, labels_hbm,
             boxout_hbm, qidx_hbm, row_v, hist_v, ckey_v, cidx_v,
             fkey_v, fidx_v, skey_v, sidx_v, sc_v, lb_v, qi_v, bx_v,
             brow_v, wh_v):
    wid = lax.axis_index("s") * 2 + lax.axis_index("c")
    lane = _lane()
    ones = jnp.ones((16,), _I32)
    zeros16i = jnp.zeros((16,), _I32)

    def clear_hist():
        @plsc.parallel_loop(0, 1024, unroll=4)
        def _(i):
            hist_v[pl.ds(i * 16, 16)] = zeros16i

    def lanesum(b):
        v = hist_v[pl.ds(b * 16, 16)]
        v = v + hist_v[pl.ds(4096 + b * 16, 16)]
        v = v + hist_v[pl.ds(8192 + b * 16, 16)]
        v = v + hist_v[pl.ds(12288 + b * 16, 16)]
        return jnp.sum(v)

    def bin_walk(g0):
        def cond(st):
            _, g, c = st
            return g + c < K_OUT

        def body(st):
            b, g, c = st
            b2 = b - 1
            return b2, g + c, lanesum(b2)

        b0 = jnp.int32(255)
        return lax.while_loop(cond, body, (b0, g0, lanesum(b0)))

    def do_row(r_i, _):
        r = wid * ROWS_PER_W + r_i
        s = (r & 1) * 4
        start = pl.multiple_of(r * N_REAL - s, 8)
        with jax.named_scope("dma_in"):
            pltpu.sync_copy(bits_hbm.at[pl.ds(start, N_DMA)],
                            row_v.at[pl.ds(0, N_DMA)])
            pltpu.sync_copy(boxes_hbm.at[r], brow_v)
            pltpu.sync_copy(wh_hbm.at[r], wh_v)

        clear_hist()

        def hist_one(i, mask):
            bits = row_v[pl.ds(i * 16, 16)]
            key = _keys_of(bits)
            if mask is not None:
                key = jnp.where(mask, key, jnp.uint32(0))
            row_v[pl.ds(i * 16, 16)] = key
            idxv = (plsc.bitcast(key >> jnp.uint32(24), _I32) << 4) | lane
            idxv = idxv | ((i & 3) << 12)
            if mask is None:
                plsc.addupdate_scatter(hist_v, [idxv], ones)
            else:
                plsc.addupdate_scatter(hist_v, [idxv], ones, mask=mask)

        with jax.named_scope("histA"):
            hist_one(jnp.int32(0), lane >= s)
            hist_one(jnp.int32(NV - 2), lane < (12 + s))
            hist_one(jnp.int32(NV - 3), None)
            row_v[pl.ds((NV - 1) * 16, 16)] = jnp.zeros((16,), _U32)

            @plsc.parallel_loop(1, NV - 3, unroll=4)
            def _(i):
                hist_one(i, None)

        b, g, c = bin_walk(jnp.int32(0))
        t_lo = b.astype(_U32) << jnp.uint32(24)
        n_ge = g + c

        def refine_cond(st):
            lvl, _, _, n = st
            return (n > CAP) & (lvl < 3)

        def refine_body(st):
            lvl, t, g_in, _ = st
            lvl2 = lvl + 1
            shift = jnp.uint32(24) - jnp.uint32(8) * lvl2.astype(_U32)
            hi = shift + jnp.uint32(8)
            clear_hist()

            def histl(i, _2):
                key = row_v[pl.ds(i * 16, 16)]
                match = (key >> hi) == (t >> hi)
                binv = (key >> shift) & jnp.uint32(0xFF)
                idxv = (plsc.bitcast(binv, _I32) << 4) | lane
                plsc.addupdate_scatter(hist_v, [idxv], ones, mask=match)
                return 0

            lax.fori_loop(0, NV, histl, 0)
            b2, g2, c2 = bin_walk(g_in)
            t2 = t | (b2.astype(_U32) << shift)
            return lvl2, t2, g2, g2 + c2

        lvl_f, t_lo, g, n_ge = lax.while_loop(
            refine_cond, refine_body, (jnp.int32(0), t_lo, g, n_ge))

        exact_from_row = n_ge > CAP

        def compact_exact(src_key, src_idx, nvec, m_src, t, e_need,
                          from_row):
            def body(i, st):
                off, eq_taken = st
                key = src_key[pl.ds(i * 16, 16)]
                if from_row:
                    idxv = lane + i * 16 - s
                    valid = None
                    m_gt = key > t
                    m_eq = key == t
                else:
                    idxv = src_idx[pl.ds(i * 16, 16)]
                    valid = (lane + i * 16) < m_src
                    m_gt = (key > t) & valid
                    m_eq = (key == t) & valid
                eqc = plsc.cumsum(m_eq.astype(_I32))
                take = m_eq & ((eq_taken + eqc) <= e_need)
                m = m_gt | take
                plsc.store_compressed(fkey_v.at[pl.ds(off, 16)],
                                      plsc.bitcast(key, _I32), mask=m)
                plsc.store_compressed(fidx_v.at[pl.ds(off, 16)], idxv,
                                      mask=m)
                return off + _popcount(m), eq_taken + _popcount(take)

            plsc.parallel_loop(0, nvec, unroll=2,
                               carry=(jnp.int32(0), jnp.int32(0)))(body)

        def from_row_exact(_2):
            compact_exact(row_v, None, jnp.int32(NV), jnp.int32(0), t_lo,
                          K_OUT - g, True)
            return 0

        def via_candidates(_2):
            def cb(i, off):
                key = row_v[pl.ds(i * 16, 16)]
                m = key >= t_lo
                plsc.store_compressed(ckey_v.at[pl.ds(off, 16)], key,
                                      mask=m)
                idxv = lane + i * 16 - s
                plsc.store_compressed(cidx_v.at[pl.ds(off, 16)], idxv,
                                      mask=m)
                return off + _popcount(m)

            m_cand = plsc.parallel_loop(0, NV, unroll=4,
                                        carry=jnp.int32(0))(cb)
            mv = (m_cand + 15) >> 4

            t = t_lo
            g2 = g
            c2 = jnp.int32(0)
            for l in range(1, 4):
                shift = jnp.uint32(24 - 8 * l)
                hi = jnp.uint32(32 - 8 * l)
                clear_hist()
                t_cur = t

                def histc(i, _3, shift=shift, hi=hi, t_cur=t_cur):
                    key = ckey_v[pl.ds(i * 16, 16)]
                    valid = (lane + i * 16) < m_cand
                    match = valid & ((key >> hi) == (t_cur >> hi))
                    binv = (key >> shift) & jnp.uint32(0xFF)
                    idxv = (plsc.bitcast(binv, _I32) << 4) | lane
                    idxv = idxv | ((i & 3) << 12)
                    plsc.addupdate_scatter(hist_v, [idxv], ones,
                                           mask=match)
                    return 0

                lax.fori_loop(0, mv, histc, 0)
                b3, g2, c2 = bin_walk(g2)
                t = t | (b3.astype(_U32) << shift)

            e_need = K_OUT - g2
            compact_exact(ckey_v, cidx_v, mv, m_cand, t, e_need, False)
            return 0

        with jax.named_scope("compact_select"):
            lax.cond(exact_from_row, from_row_exact, via_candidates, 0)

        nv_f = K_PAD // 16

        with jax.named_scope("rank"):
            pad_pos = (lane & 3) + K_OUT
            pad_m = lane < 4
            plsc.store_scatter(fkey_v, [pad_pos], zeros16i, mask=pad_m)
            plsc.store_scatter(fidx_v, [pad_pos],
                               jnp.full((16,), 0x7FFFFFFF, _I32), mask=pad_m)
            skey_v[pl.ds(K_PAD - 16, 16)] = zeros16i
            sidx_v[pl.ds(K_PAD - 16, 16)] = zeros16i

            def rank_body(j, ranks):
                jb = zeros16i + j
                kj = plsc.bitcast(plsc.load_gather(fkey_v, [jb]), _U32)
                ij = plsc.load_gather(fidx_v, [jb])
                out = []
                for v in range(nv_f):
                    kv = plsc.bitcast(fkey_v[pl.ds(v * 16, 16)], _U32)
                    iv = fidx_v[pl.ds(v * 16, 16)]
                    beat = (kj > kv) | ((kj == kv) & (ij < iv))
                    out.append(ranks[v] + beat.astype(_I32))
                return tuple(out)

            ranks = plsc.parallel_loop(
                0, K_OUT, unroll=2,
                carry=tuple(zeros16i for _ in range(nv_f)))(rank_body)

            for v in range(nv_f):
                valid = (lane + v * 16) < K_OUT
                plsc.store_scatter(skey_v, [ranks[v]],
                                   fkey_v[pl.ds(v * 16, 16)], mask=valid)
                plsc.store_scatter(sidx_v, [ranks[v]],
                                   fidx_v[pl.ds(v * 16, 16)], mask=valid)

        decode_scope = jax.named_scope("decode")
        decode_scope.__enter__()
        wvec = wh_v[pl.ds(0, 16)]
        hvec = wh_v[pl.ds(16, 16)]
        lane4 = lane * 4

        @plsc.parallel_loop(0, nv_f, unroll=2)
        def _decode(g_i):
            key = plsc.bitcast(skey_v[pl.ds(g_i * 16, 16)], _U32)
            msb = key >= jnp.uint32(0x80000000)
            bits = jnp.where(msb, key ^ jnp.uint32(0x80000000), ~key)
            x = plsc.bitcast(bits, jnp.float32)
            score = 1.0 / (1.0 + jnp.exp(-x))
            idx = sidx_v[pl.ds(g_i * 16, 16)]
            q = lax.div(idx, jnp.int32(NUM_C))
            label = idx - q * NUM_C
            q4 = q * 4
            cx = plsc.load_gather(brow_v, [q4])
            cy = plsc.load_gather(brow_v, [q4 + 1])
            w = plsc.load_gather(brow_v, [q4 + 2])
            h = plsc.load_gather(brow_v, [q4 + 3])
            hw = 0.5 * w
            hh = 0.5 * h
            sc_v[pl.ds(g_i * 16, 16)] = score
            lb_v[pl.ds(g_i * 16, 16)] = label
            qi_v[pl.ds(g_i * 16, 16)] = q
            base = g_i * 64
            plsc.store_scatter(bx_v, [lane4 + base], (cx - hw) * wvec)
            plsc.store_scatter(bx_v, [lane4 + base + 1], (cy - hh) * hvec)
            plsc.store_scatter(bx_v, [lane4 + base + 2], (cx + hw) * wvec)
            plsc.store_scatter(bx_v, [lane4 + base + 3], (cy + hh) * hvec)

        decode_scope.__exit__(None, None, None)
        with jax.named_scope("dma_out"):
            pltpu.sync_copy(sc_v, scores_hbm.at[r])
            pltpu.sync_copy(lb_v, labels_hbm.at[r])
            pltpu.sync_copy(qi_v, qidx_hbm.at[r])
            pltpu.sync_copy(bx_v, boxout_hbm.at[r])
        return 0

    lax.fori_loop(0, ROWS_PER_W, do_row, 0)


@jax.jit
def _topk_sc(bits, boxes_flat, wh):
    b = 128
    mesh = plsc.VectorSubcoreMesh(core_axis_name="c", subcore_axis_name="s",
                                  num_cores=2, num_subcores=16)
    out_type = (
        jax.ShapeDtypeStruct((b, K_PAD), jnp.float32),
        jax.ShapeDtypeStruct((b, K_PAD), jnp.int32),
        jax.ShapeDtypeStruct((b, K_PAD * 4), jnp.float32),
        jax.ShapeDtypeStruct((b, K_PAD), jnp.int32),
    )
    scratch = [
        pltpu.VMEM((N_BUF,), _U32),
        pltpu.VMEM((4 * 4096,), _I32),
        pltpu.VMEM((CAP + 16,), _U32),
        pltpu.VMEM((CAP + 16,), _I32),
        pltpu.VMEM((K_PAD,), _I32),
        pltpu.VMEM((K_PAD,), _I32),
        pltpu.VMEM((K_PAD,), _I32),
        pltpu.VMEM((K_PAD,), _I32),
        pltpu.VMEM((K_PAD,), jnp.float32),
        pltpu.VMEM((K_PAD,), _I32),
        pltpu.VMEM((K_PAD,), _I32),
        pltpu.VMEM((K_PAD * 4,), jnp.float32),
        pltpu.VMEM((NUM_Q * 4,), jnp.float32),
        pltpu.VMEM((32,), jnp.float32),
    ]
    f = pl.kernel(_tk_body, out_type=out_type, mesh=mesh,
                  scratch_types=scratch,
                  compiler_params=pltpu.CompilerParams(
                      needs_layout_passes=False))
    return f(bits, boxes_flat, wh)


def kernel(pred_logits, pred_boxes, target_sizes):
    b, q, c = pred_logits.shape
    bits = lax.bitcast_convert_type(pred_logits, _U32).reshape(b * q * c)
    boxes_flat = pred_boxes.reshape(b, q * 4)
    ts = target_sizes.astype(jnp.float32)
    wv = jnp.broadcast_to(ts[:, 1:2], (b, 16))
    hv = jnp.broadcast_to(ts[:, 0:1], (b, 16))
    wh = jnp.concatenate([wv, hv], axis=1)
    scores, labels, boxes, qidx = _topk_sc(bits, boxes_flat, wh)
    boxes = boxes[:, : K_OUT * 4].reshape(b, K_OUT, 4)
    return (scores[:, :K_OUT], labels[:, :K_OUT], boxes, qidx[:, :K_OUT])


# speedup vs baseline: 1.0260x; 1.0260x over previous
import jax
import jax.numpy as jnp
from jax import lax
from jax.experimental import pallas as pl
from jax.experimental.pallas import tpu as pltpu
from jax.experimental.pallas import tpu_sc as plsc

NUM_Q = 900
NUM_C = 91
N_REAL = NUM_Q * NUM_C
N_BUF = 81920
NV = N_BUF // 16
N_DMA = 81904
K_OUT = 300
K_PAD = 304
CAP = 4096
ROWS_PER_W = 4

_I32 = jnp.int32
_U32 = jnp.uint32


def _lane():
    return lax.iota(_I32, 16)


def _popcount(mask):
    return plsc.all_reduce_population_count(mask)[0]


def _keys_of(bits):
    m = plsc.bitcast(plsc.bitcast(bits, _I32) >> 31, _U32)
    return bits ^ (m | jnp.uint32(0x80000000))


def _tk_body(bits_hbm, boxes_hbm, wh_hbm, scores_hbm, labels_hbm,
             boxout_hbm, qidx_hbm, row_v, hist_v, ckey_v, cidx_v,
             fkey_v, fidx_v, skey_v, sidx_v, sc_v, lb_v, qi_v, bx_v,
             brow_v, wh_v):
    wid = lax.axis_index("s") * 2 + lax.axis_index("c")
    lane = _lane()
    ones = jnp.ones((16,), _I32)
    zeros16i = jnp.zeros((16,), _I32)

    def clear_hist():
        @plsc.parallel_loop(0, 1024, unroll=4)
        def _(i):
            hist_v[pl.ds(i * 16, 16)] = zeros16i

    def lanesum(b):
        v = hist_v[pl.ds(b * 16, 16)]
        v = v + hist_v[pl.ds(4096 + b * 16, 16)]
        v = v + hist_v[pl.ds(8192 + b * 16, 16)]
        v = v + hist_v[pl.ds(12288 + b * 16, 16)]
        return jnp.sum(v)

    def bin_walk(g0):
        def cond(st):
            _, g, c = st
            return g + c < K_OUT

        def body(st):
            b, g, c = st
            b2 = b - 1
            return b2, g + c, lanesum(b2)

        b0 = jnp.int32(255)
        return lax.while_loop(cond, body, (b0, g0, lanesum(b0)))

    def do_row(r_i, _):
        r = wid * ROWS_PER_W + r_i
        s = (r & 1) * 4
        start = pl.multiple_of(r * N_REAL - s, 8)
        with jax.named_scope("dma_in"):
            pltpu.sync_copy(bits_hbm.at[pl.ds(start, N_DMA)],
                            row_v.at[pl.ds(0, N_DMA)])
            pltpu.sync_copy(boxes_hbm.at[r], brow_v)
            pltpu.sync_copy(wh_hbm.at[r], wh_v)

        clear_hist()

        def hist_one(i, mask):
            bits = plsc.bitcast(row_v[pl.ds(i * 16, 16)], _U32)
            key = _keys_of(bits)
            if mask is not None:
                key = jnp.where(mask, key, jnp.uint32(0))
            row_v[pl.ds(i * 16, 16)] = plsc.bitcast(key, jnp.float32)
            idxv = (plsc.bitcast(key >> jnp.uint32(24), _I32) << 4) | lane
            idxv = idxv | ((i & 3) << 12)
            if mask is None:
                plsc.addupdate_scatter(hist_v, [idxv], ones)
            else:
                plsc.addupdate_scatter(hist_v, [idxv], ones, mask=mask)

        with jax.named_scope("histA"):
            hist_one(jnp.int32(0), lane >= s)
            hist_one(jnp.int32(NV - 2), lane < (12 + s))
            hist_one(jnp.int32(NV - 3), None)
            row_v[pl.ds((NV - 1) * 16, 16)] = jnp.zeros((16,), jnp.float32)

            @plsc.parallel_loop(1, NV - 3, unroll=4)
            def _(i):
                hist_one(i, None)

        b, g, c = bin_walk(jnp.int32(0))
        t_lo = b.astype(_U32) << jnp.uint32(24)
        n_ge = g + c

        def refine_cond(st):
            lvl, _, _, n = st
            return (n > CAP) & (lvl < 3)

        def refine_body(st):
            lvl, t, g_in, _ = st
            lvl2 = lvl + 1
            shift = jnp.uint32(24) - jnp.uint32(8) * lvl2.astype(_U32)
            hi = shift + jnp.uint32(8)
            clear_hist()

            def histl(i, _2):
                key = plsc.bitcast(row_v[pl.ds(i * 16, 16)], _U32)
                match = (key >> hi) == (t >> hi)
                binv = (key >> shift) & jnp.uint32(0xFF)
                idxv = (plsc.bitcast(binv, _I32) << 4) | lane
                plsc.addupdate_scatter(hist_v, [idxv], ones, mask=match)
                return 0

            lax.fori_loop(0, NV, histl, 0)
            b2, g2, c2 = bin_walk(g_in)
            t2 = t | (b2.astype(_U32) << shift)
            return lvl2, t2, g2, g2 + c2

        lvl_f, t_lo, g, n_ge = lax.while_loop(
            refine_cond, refine_body, (jnp.int32(0), t_lo, g, n_ge))

        exact_from_row = n_ge > CAP

        def compact_exact(src_key, src_idx, nvec, m_src, t, e_need,
                          from_row):
            def body(i, st):
                off, eq_taken = st
                key = src_key[pl.ds(i * 16, 16)]
                if from_row:
                    key = plsc.bitcast(key, _U32)
                    idxv = lane + i * 16 - s
                    valid = None
                    m_gt = key > t
                    m_eq = key == t
                else:
                    idxv = src_idx[pl.ds(i * 16, 16)]
                    valid = (lane + i * 16) < m_src
                    m_gt = (key > t) & valid
                    m_eq = (key == t) & valid
                eqc = plsc.cumsum(m_eq.astype(_I32))
                take = m_eq & ((eq_taken + eqc) <= e_need)
                m = m_gt | take
                plsc.store_compressed(fkey_v.at[pl.ds(off, 16)],
                                      plsc.bitcast(key, _I32), mask=m)
                plsc.store_compressed(fidx_v.at[pl.ds(off, 16)], idxv,
                                      mask=m)
                return off + _popcount(m), eq_taken + _popcount(take)

            plsc.parallel_loop(0, nvec, unroll=2,
                               carry=(jnp.int32(0), jnp.int32(0)))(body)

        def from_row_exact(_2):
            compact_exact(row_v, None, jnp.int32(NV), jnp.int32(0), t_lo,
                          K_OUT - g, True)
            return 0

        def via_candidates(_2):
            def cb(i, off):
                key = plsc.bitcast(row_v[pl.ds(i * 16, 16)], _U32)
                m = key >= t_lo
                plsc.store_compressed(ckey_v.at[pl.ds(off, 16)], key,
                                      mask=m)
                idxv = lane + i * 16 - s
                plsc.store_compressed(cidx_v.at[pl.ds(off, 16)], idxv,
                                      mask=m)
                return off + _popcount(m)

            m_cand = plsc.parallel_loop(0, NV, unroll=4,
                                        carry=jnp.int32(0))(cb)
            mv = (m_cand + 15) >> 4

            t = t_lo
            g2 = g
            c2 = jnp.int32(0)
            for l in range(1, 4):
                shift = jnp.uint32(24 - 8 * l)
                hi = jnp.uint32(32 - 8 * l)
                clear_hist()
                t_cur = t

                def histc(i, _3, shift=shift, hi=hi, t_cur=t_cur):
                    key = ckey_v[pl.ds(i * 16, 16)]
                    valid = (lane + i * 16) < m_cand
                    match = valid & ((key >> hi) == (t_cur >> hi))
                    binv = (key >> shift) & jnp.uint32(0xFF)
                    idxv = (plsc.bitcast(binv, _I32) << 4) | lane
                    idxv = idxv | ((i & 3) << 12)
                    plsc.addupdate_scatter(hist_v, [idxv], ones,
                                           mask=match)
                    return 0

                lax.fori_loop(0, mv, histc, 0)
                b3, g2, c2 = bin_walk(g2)
                t = t | (b3.astype(_U32) << shift)

            e_need = K_OUT - g2
            compact_exact(ckey_v, cidx_v, mv, m_cand, t, e_need, False)
            return 0

        with jax.named_scope("compact_select"):
            lax.cond(exact_from_row, from_row_exact, via_candidates, 0)

        nv_f = K_PAD // 16

        with jax.named_scope("rank"):
            pad_pos = (lane & 3) + K_OUT
            pad_m = lane < 4
            plsc.store_scatter(fkey_v, [pad_pos], zeros16i, mask=pad_m)
            plsc.store_scatter(fidx_v, [pad_pos],
                               jnp.full((16,), 0x7FFFFFFF, _I32), mask=pad_m)
            skey_v[pl.ds(K_PAD - 16, 16)] = zeros16i
            sidx_v[pl.ds(K_PAD - 16, 16)] = zeros16i

            def rank_body(j, ranks):
                jb = zeros16i + j
                kj = plsc.bitcast(plsc.load_gather(fkey_v, [jb]), _U32)
                ij = plsc.load_gather(fidx_v, [jb])
                out = []
                for v in range(nv_f):
                    kv = plsc.bitcast(fkey_v[pl.ds(v * 16, 16)], _U32)
                    iv = fidx_v[pl.ds(v * 16, 16)]
                    beat = (kj > kv) | ((kj == kv) & (ij < iv))
                    out.append(ranks[v] + beat.astype(_I32))
                return tuple(out)

            ranks = plsc.parallel_loop(
                0, K_OUT, unroll=2,
                carry=tuple(zeros16i for _ in range(nv_f)))(rank_body)

            for v in range(nv_f):
                valid = (lane + v * 16) < K_OUT
                plsc.store_scatter(skey_v, [ranks[v]],
                                   fkey_v[pl.ds(v * 16, 16)], mask=valid)
                plsc.store_scatter(sidx_v, [ranks[v]],
                                   fidx_v[pl.ds(v * 16, 16)], mask=valid)

        decode_scope = jax.named_scope("decode")
        decode_scope.__enter__()
        wvec = wh_v[pl.ds(0, 16)]
        hvec = wh_v[pl.ds(16, 16)]
        lane4 = lane * 4

        @plsc.parallel_loop(0, nv_f, unroll=2)
        def _decode(g_i):
            key = plsc.bitcast(skey_v[pl.ds(g_i * 16, 16)], _U32)
            msb = key >= jnp.uint32(0x80000000)
            bits = jnp.where(msb, key ^ jnp.uint32(0x80000000), ~key)
            x = plsc.bitcast(bits, jnp.float32)
            score = 1.0 / (1.0 + jnp.exp(-x))
            idx = sidx_v[pl.ds(g_i * 16, 16)]
            q = lax.div(idx, jnp.int32(NUM_C))
            label = idx - q * NUM_C
            q4 = q * 4
            cx = plsc.load_gather(brow_v, [q4])
            cy = plsc.load_gather(brow_v, [q4 + 1])
            w = plsc.load_gather(brow_v, [q4 + 2])
            h = plsc.load_gather(brow_v, [q4 + 3])
            hw = 0.5 * w
            hh = 0.5 * h
            sc_v[pl.ds(g_i * 16, 16)] = score
            lb_v[pl.ds(g_i * 16, 16)] = label
            qi_v[pl.ds(g_i * 16, 16)] = q
            base = g_i * 64
            plsc.store_scatter(bx_v, [lane4 + base], (cx - hw) * wvec)
            plsc.store_scatter(bx_v, [lane4 + base + 1], (cy - hh) * hvec)
            plsc.store_scatter(bx_v, [lane4 + base + 2], (cx + hw) * wvec)
            plsc.store_scatter(bx_v, [lane4 + base + 3], (cy + hh) * hvec)

        decode_scope.__exit__(None, None, None)
        with jax.named_scope("dma_out"):
            pltpu.sync_copy(sc_v, scores_hbm.at[r])
            pltpu.sync_copy(lb_v, labels_hbm.at[r])
            pltpu.sync_copy(qi_v, qidx_hbm.at[r])
            pltpu.sync_copy(bx_v, boxout_hbm.at[r])
        return 0

    lax.fori_loop(0, ROWS_PER_W, do_row, 0)


@jax.jit
def _topk_sc(bits, boxes_flat, wh):
    b = 128
    mesh = plsc.VectorSubcoreMesh(core_axis_name="c", subcore_axis_name="s",
                                  num_cores=2, num_subcores=16)
    out_type = (
        jax.ShapeDtypeStruct((b, K_PAD), jnp.float32),
        jax.ShapeDtypeStruct((b, K_PAD), jnp.int32),
        jax.ShapeDtypeStruct((b, K_PAD * 4), jnp.float32),
        jax.ShapeDtypeStruct((b, K_PAD), jnp.int32),
    )
    scratch = [
        pltpu.VMEM((N_BUF,), jnp.float32),
        pltpu.VMEM((4 * 4096,), _I32),
        pltpu.VMEM((CAP + 16,), _U32),
        pltpu.VMEM((CAP + 16,), _I32),
        pltpu.VMEM((K_PAD,), _I32),
        pltpu.VMEM((K_PAD,), _I32),
        pltpu.VMEM((K_PAD,), _I32),
        pltpu.VMEM((K_PAD,), _I32),
        pltpu.VMEM((K_PAD,), jnp.float32),
        pltpu.VMEM((K_PAD,), _I32),
        pltpu.VMEM((K_PAD,), _I32),
        pltpu.VMEM((K_PAD * 4,), jnp.float32),
        pltpu.VMEM((NUM_Q * 4,), jnp.float32),
        pltpu.VMEM((32,), jnp.float32),
    ]
    f = pl.kernel(_tk_body, out_type=out_type, mesh=mesh,
                  scratch_types=scratch,
                  compiler_params=pltpu.CompilerParams(
                      needs_layout_passes=False))
    return f(bits, boxes_flat, wh)


def kernel(pred_logits, pred_boxes, target_sizes):
    b, q, c = pred_logits.shape
    bits = pred_logits.reshape(b * q * c)
    boxes_flat = pred_boxes.reshape(b, q * 4)
    ts = target_sizes.astype(jnp.float32)
    wv = jnp.broadcast_to(ts[:, 1:2], (b, 16))
    hv = jnp.broadcast_to(ts[:, 0:1], (b, 16))
    wh = jnp.concatenate([wv, hv], axis=1)
    scores, labels, boxes, qidx = _topk_sc(bits, boxes_flat, wh)
    boxes = boxes[:, : K_OUT * 4].reshape(b, K_OUT, 4)
    return (scores[:, :K_OUT], labels[:, :K_OUT], boxes, qidx[:, :K_OUT])
